# baseline (speedup 1.0000x reference)
import functools

import jax
import jax.numpy as jnp
from jax import lax
from jax.experimental import pallas as pl
from jax.experimental.pallas import tpu as pltpu
from jax.experimental.pallas import tpu_sc as plsc

NC = 2
NS = 16
NW = NC * NS

S = 128


@functools.partial(jax.jit, static_argnames=("N", "Q", "D"))
def _sc_gather(idxT, weight, N, Q, D):
    nb_w = N // NW
    C = nb_w // S
    T = Q * C

    mesh = plsc.VectorSubcoreMesh(core_axis_name="c", subcore_axis_name="s")

    @functools.partial(
        pl.kernel,
        mesh=mesh,
        out_type=jax.ShapeDtypeStruct((N, Q, D), jnp.float32),
        scratch_types=[
            pltpu.VMEM((C, Q, S), jnp.int32),
            pltpu.VMEM((S, D), jnp.float32),
            pltpu.VMEM((S, D), jnp.float32),
            pltpu.SemaphoreType.DMA,
            pltpu.SemaphoreType.DMA,
            pltpu.SemaphoreType.DMA,
            pltpu.SemaphoreType.DMA,
        ],
        compiler_params=pltpu.CompilerParams(use_tc_tiling_on_sc=False),
    )
    def k(idxT_hbm, w_hbm, out_hbm, idx_v, buf0, buf1, sg0, sg1, sw0, sw1):
        wid = lax.axis_index("s") * NC + lax.axis_index("c")
        b0w = wid * nb_w
        for c in range(C):
            pltpu.sync_copy(
                idxT_hbm.at[:, pl.ds(b0w + c * S, S)], idx_v.at[c]
            )

        def fire_gather(t, buf, sem):
            q = t // C
            c = lax.rem(t, C)
            pltpu.async_copy(w_hbm.at[idx_v.at[c, q]], buf, sem)

        def drain_gather(buf, sem):
            pltpu.make_async_copy(out_hbm.at[pl.ds(0, S), 0], buf, sem).wait()

        def fire_write(t, buf, sem):
            q = t // C
            c = lax.rem(t, C)
            pltpu.async_copy(
                buf, out_hbm.at[pl.ds(b0w + c * S, S), q], sem
            )

        def drain_write(buf, sem):
            pltpu.make_async_copy(out_hbm.at[pl.ds(0, S), 0], buf, sem).wait()

        fire_gather(0, buf0, sg0)

        fire_gather(1, buf1, sg1)
        drain_gather(buf0, sg0)
        fire_write(0, buf0, sw0)
        drain_write(buf0, sw0)
        fire_gather(2, buf0, sg0)
        drain_gather(buf1, sg1)
        fire_write(1, buf1, sw1)

        def body(i, carry):
            t = 2 * i
            drain_write(buf1, sw1)
            fire_gather(t + 1, buf1, sg1)
            drain_gather(buf0, sg0)
            fire_write(t, buf0, sw0)
            drain_write(buf0, sw0)
            fire_gather(t + 2, buf0, sg0)
            drain_gather(buf1, sg1)
            fire_write(t + 1, buf1, sw1)
            return carry

        lax.fori_loop(1, T // 2 - 1, body, 0)

        drain_write(buf1, sw1)
        fire_gather(T - 1, buf1, sg1)
        drain_gather(buf0, sg0)
        fire_write(T - 2, buf0, sw0)
        drain_gather(buf1, sg1)
        fire_write(T - 1, buf1, sw1)
        drain_write(buf0, sw0)
        drain_write(buf1, sw1)

    return k(idxT, weight)


def kernel(idx, weight):
    N, Q = idx.shape
    D = weight.shape[1]
    idxT = jnp.transpose(idx.astype(jnp.int32))
    return _sc_gather(idxT, weight, N, Q, D)


# speedup vs baseline: 1.0678x; 1.0678x over previous
import functools

import jax
import jax.numpy as jnp
from jax import lax
from jax.experimental import pallas as pl
from jax.experimental.pallas import tpu as pltpu
from jax.experimental.pallas import tpu_sc as plsc

NC = 2
NS = 16
NW = NC * NS

S = 128


@functools.partial(jax.jit, static_argnames=("N", "Q", "D"))
def _sc_gather(idx3, weight, N, Q, D):
    nb_w = N // NW
    C = nb_w // S
    T = Q * C

    mesh = plsc.VectorSubcoreMesh(core_axis_name="c", subcore_axis_name="s")

    @functools.partial(
        pl.kernel,
        mesh=mesh,
        out_type=jax.ShapeDtypeStruct((Q, N, D), jnp.float32),
        scratch_types=[
            pltpu.VMEM((Q, C, S), jnp.int32),
            pltpu.VMEM((S, D), jnp.float32),
            pltpu.VMEM((S, D), jnp.float32),
            pltpu.SemaphoreType.DMA,
            pltpu.SemaphoreType.DMA,
            pltpu.SemaphoreType.DMA,
            pltpu.SemaphoreType.DMA,
        ],
        compiler_params=pltpu.CompilerParams(use_tc_tiling_on_sc=False),
    )
    def k(idx_hbm, w_hbm, outT_hbm, idx_v, buf0, buf1, sg0, sg1, sw0, sw1):
        wid = lax.axis_index("s") * NC + lax.axis_index("c")
        b0w = wid * nb_w
        pltpu.sync_copy(idx_hbm.at[:, pl.ds(wid * C, C), :], idx_v)

        def fire_gather(t, buf, sem):
            q = t // C
            c = lax.rem(t, C)
            pltpu.async_copy(w_hbm.at[idx_v.at[q, c]], buf, sem)

        def drain_gather(buf, sem):
            pltpu.make_async_copy(
                outT_hbm.at[0, pl.ds(0, S)], buf, sem
            ).wait()

        def fire_write(t, buf, sem):
            q = t // C
            c = lax.rem(t, C)
            pltpu.async_copy(
                buf, outT_hbm.at[q, pl.ds(b0w + c * S, S)], sem
            )

        def drain_write(buf, sem):
            pltpu.make_async_copy(
                outT_hbm.at[0, pl.ds(0, S)], buf, sem
            ).wait()

        fire_gather(0, buf0, sg0)

        fire_gather(1, buf1, sg1)
        drain_gather(buf0, sg0)
        fire_write(0, buf0, sw0)
        drain_write(buf0, sw0)
        fire_gather(2, buf0, sg0)
        drain_gather(buf1, sg1)
        fire_write(1, buf1, sw1)

        def body(i, carry):
            t = 2 * i
            drain_write(buf1, sw1)
            fire_gather(t + 1, buf1, sg1)
            drain_gather(buf0, sg0)
            fire_write(t, buf0, sw0)
            drain_write(buf0, sw0)
            fire_gather(t + 2, buf0, sg0)
            drain_gather(buf1, sg1)
            fire_write(t + 1, buf1, sw1)
            return carry

        lax.fori_loop(1, T // 2 - 1, body, 0)

        drain_write(buf1, sw1)
        fire_gather(T - 1, buf1, sg1)
        drain_gather(buf0, sg0)
        fire_write(T - 2, buf0, sw0)
        drain_gather(buf1, sg1)
        fire_write(T - 1, buf1, sw1)
        drain_write(buf0, sw0)
        drain_write(buf1, sw1)

    return k(idx3, weight)


def kernel(idx, weight):
    N, Q = idx.shape
    D = weight.shape[1]
    idx3 = jnp.reshape(jnp.transpose(idx.astype(jnp.int32)), (Q, N // S, S))
    outT = _sc_gather(idx3, weight, N, Q, D)
    return jnp.transpose(outT, (1, 0, 2))


# speedup vs baseline: 1.1273x; 1.0557x over previous
import functools

import jax
import jax.numpy as jnp
from jax import lax
from jax.experimental import pallas as pl
from jax.experimental.pallas import tpu as pltpu
from jax.experimental.pallas import tpu_sc as plsc

NC = 2
NS = 16
NW = NC * NS

S = 128


@functools.partial(jax.jit, static_argnames=("N", "Q", "D"))
def _sc_gather(idx3, weight, N, Q, D):
    nb_w = N // NW
    C = nb_w // S
    T = Q

    mesh = plsc.VectorSubcoreMesh(core_axis_name="c", subcore_axis_name="s")

    @functools.partial(
        pl.kernel,
        mesh=mesh,
        out_type=jax.ShapeDtypeStruct((Q, N, D), jnp.float32),
        scratch_types=[
            pltpu.VMEM((Q, C, S), jnp.int32),
            pltpu.VMEM((nb_w, D), jnp.float32),
            pltpu.VMEM((nb_w, D), jnp.float32),
            pltpu.SemaphoreType.DMA,
            pltpu.SemaphoreType.DMA,
            pltpu.SemaphoreType.DMA,
            pltpu.SemaphoreType.DMA,
        ],
        compiler_params=pltpu.CompilerParams(use_tc_tiling_on_sc=False),
    )
    def k(idx_hbm, w_hbm, outT_hbm, idx_v, buf0, buf1, sg0, sg1, sw0, sw1):
        wid = lax.axis_index("s") * NC + lax.axis_index("c")
        b0w = wid * nb_w
        pltpu.sync_copy(idx_hbm.at[:, pl.ds(wid * C, C), :], idx_v)

        def fire_gather(q, buf, sem):
            for c in range(C):
                pltpu.async_copy(
                    w_hbm.at[idx_v.at[q, c]], buf.at[pl.ds(c * S, S)], sem
                )

        def drain_gather(buf, sem):
            pltpu.make_async_copy(
                outT_hbm.at[0, pl.ds(0, nb_w)], buf, sem
            ).wait()

        def fire_write(q, buf, sem):
            pltpu.async_copy(buf, outT_hbm.at[q, pl.ds(b0w, nb_w)], sem)

        def drain_write(buf, sem):
            pltpu.make_async_copy(
                outT_hbm.at[0, pl.ds(0, nb_w)], buf, sem
            ).wait()

        fire_gather(0, buf0, sg0)

        fire_gather(1, buf1, sg1)
        drain_gather(buf0, sg0)
        fire_write(0, buf0, sw0)
        drain_write(buf0, sw0)
        fire_gather(2, buf0, sg0)
        drain_gather(buf1, sg1)
        fire_write(1, buf1, sw1)

        def body(i, carry):
            t = 2 * i
            drain_write(buf1, sw1)
            fire_gather(t + 1, buf1, sg1)
            drain_gather(buf0, sg0)
            fire_write(t, buf0, sw0)
            drain_write(buf0, sw0)
            fire_gather(t + 2, buf0, sg0)
            drain_gather(buf1, sg1)
            fire_write(t + 1, buf1, sw1)
            return carry

        lax.fori_loop(1, T // 2 - 1, body, 0)

        drain_write(buf1, sw1)
        fire_gather(T - 1, buf1, sg1)
        drain_gather(buf0, sg0)
        fire_write(T - 2, buf0, sw0)
        drain_gather(buf1, sg1)
        fire_write(T - 1, buf1, sw1)
        drain_write(buf0, sw0)
        drain_write(buf1, sw1)

    return k(idx3, weight)


def kernel(idx, weight):
    N, Q = idx.shape
    D = weight.shape[1]
    idx3 = jnp.reshape(jnp.transpose(idx.astype(jnp.int32)), (Q, N // S, S))
    outT = _sc_gather(idx3, weight, N, Q, D)
    return jnp.transpose(outT, (1, 0, 2))
